# baseline (speedup 1.0000x reference)
import functools

import jax
import jax.numpy as jnp
from jax import lax
from jax.experimental import pallas as pl
from jax.experimental.pallas import tpu as pltpu
from jax.experimental.pallas import tpu_sc as plsc

NUM_EMBEDDINGS = 1000000
EMBEDDING_DIM = 32
SHARED_DIM = 4
TABLE_DIM = 28
BATCH = 16384

_info = plsc.get_sparse_core_info()
_NC, _NS = _info.num_cores, _info.num_subcores
_NW = _NC * _NS
_BPW = BATCH // _NW
_CHUNK = 128
_NCHUNK = _BPW // _CHUNK


@functools.partial(
    pl.kernel,
    mesh=plsc.VectorSubcoreMesh(core_axis_name="c", subcore_axis_name="s"),
    out_type=jax.ShapeDtypeStruct((BATCH, EMBEDDING_DIM), jnp.float32),
    compiler_params=pltpu.CompilerParams(
        use_tc_tiling_on_sc=False, needs_layout_passes=False
    ),
    scratch_types=[
        pltpu.VMEM((_BPW,), jnp.int32),
        pltpu.VMEM((_BPW, EMBEDDING_DIM), jnp.float32),
        pltpu.SemaphoreType.DMA,
    ],
)
def _embed_lookup(table_hbm, idx_hbm, tpl_hbm, out_hbm, idx_v, out_v, sem):
    wid = lax.axis_index("s") * _NC + lax.axis_index("c")
    base = wid * _BPW
    pltpu.sync_copy(idx_hbm.at[pl.ds(base, _BPW)], idx_v)
    pltpu.sync_copy(tpl_hbm, out_v)
    copies = []
    for c in range(_NCHUNK):
        copies.append(
            pltpu.async_copy(
                table_hbm.at[idx_v.at[pl.ds(c * _CHUNK, _CHUNK)]],
                out_v.at[pl.ds(c * _CHUNK, _CHUNK)],
                sem,
                add=True,
            )
        )
    for cp in copies:
        cp.wait()
    pltpu.sync_copy(out_v, out_hbm.at[pl.ds(base, _BPW)])


def kernel(x, embed_table, shared_embed):
    idx = x.astype(jnp.int32)
    table32 = jnp.pad(embed_table, ((0, 0), (0, SHARED_DIM)))
    row_tpl = jnp.concatenate(
        [jnp.zeros((TABLE_DIM,), jnp.float32),
         shared_embed.reshape(SHARED_DIM).astype(jnp.float32)]
    )
    tpl = jnp.broadcast_to(row_tpl[None, :], (_BPW, EMBEDDING_DIM))
    out = _embed_lookup(table32, idx, tpl)
    return out.reshape(BATCH, 1, EMBEDDING_DIM)


# speedup vs baseline: 1.4137x; 1.4137x over previous
import functools

import jax
import jax.numpy as jnp
from jax import lax
from jax.experimental import pallas as pl
from jax.experimental.pallas import tpu as pltpu
from jax.experimental.pallas import tpu_sc as plsc

NUM_EMBEDDINGS = 1000000
EMBEDDING_DIM = 32
SHARED_DIM = 4
TABLE_DIM = 28
BATCH = 16384
_SROWS = NUM_EMBEDDINGS // 4

_info = plsc.get_sparse_core_info()
_NC, _NS = _info.num_cores, _info.num_subcores
_NW = _NC * _NS
_BPW = BATCH // _NW
_CHUNK = 128
_NCHUNK = _BPW // _CHUNK
_L = 16


@functools.partial(
    pl.kernel,
    mesh=plsc.VectorSubcoreMesh(core_axis_name="c", subcore_axis_name="s"),
    out_type=jax.ShapeDtypeStruct((EMBEDDING_DIM, BATCH), jnp.float32),
    compiler_params=pltpu.CompilerParams(
        use_tc_tiling_on_sc=True, needs_layout_passes=False
    ),
    scratch_types=[
        pltpu.VMEM((_BPW,), jnp.int32),
        pltpu.VMEM((_BPW,), jnp.int32),
        pltpu.VMEM((_BPW, 128), jnp.float32),
        pltpu.VMEM((EMBEDDING_DIM, _BPW), jnp.float32),
        pltpu.SemaphoreType.DMA,
    ],
)
def _embed_lookup(t128_hbm, srow_hbm, qoff_hbm, outT_hbm,
                  srow_v, qoff_v, big_v, outT_v, sem):
    wid = lax.axis_index("s") * _NC + lax.axis_index("c")
    base = wid * _BPW
    pltpu.sync_copy(srow_hbm.at[pl.ds(base, _BPW)], srow_v)
    pltpu.sync_copy(qoff_hbm.at[pl.ds(base, _BPW)], qoff_v)
    copies = []
    for c in range(_NCHUNK):
        copies.append(
            pltpu.async_copy(
                t128_hbm.at[srow_v.at[pl.ds(c * _CHUNK, _CHUNK)]],
                big_v.at[pl.ds(c * _CHUNK, _CHUNK)],
                sem,
            )
        )

    lanes = lax.iota(jnp.int32, _L)

    def extract(i, _):
        qsplat = plsc.load_gather(qoff_v, [jnp.full((_L,), i, jnp.int32)])
        row_ids = jnp.full((_L,), i, jnp.int32)
        lo = plsc.load_gather(big_v, [row_ids, qsplat + lanes])
        hi = plsc.load_gather(big_v, [row_ids, qsplat + (lanes + _L)])
        jsplat = jnp.full((_L,), i, jnp.int32)
        plsc.store_scatter(outT_v, [lanes, jsplat], lo)
        plsc.store_scatter(outT_v, [lanes + _L, jsplat], hi)
        return _

    for c in range(_NCHUNK):
        copies[c].wait()
        lax.fori_loop(c * _CHUNK, (c + 1) * _CHUNK, extract, None)
    pltpu.sync_copy(outT_v, outT_hbm.at[:, pl.ds(base, _BPW)])


def kernel(x, embed_table, shared_embed):
    idx = x.astype(jnp.int32)
    srow = idx // 4
    qoff = (idx % 4) * EMBEDDING_DIM
    shared_col = jnp.broadcast_to(
        shared_embed.reshape(1, SHARED_DIM), (NUM_EMBEDDINGS, SHARED_DIM)
    )
    t128 = jnp.concatenate([embed_table, shared_col], axis=1).reshape(_SROWS, 128)
    outT = _embed_lookup(t128, srow, qoff)
    return outT.T.reshape(BATCH, 1, EMBEDDING_DIM)


# speedup vs baseline: 4.0754x; 2.8827x over previous
import functools

import jax
import jax.numpy as jnp
from jax import lax
from jax.experimental import pallas as pl
from jax.experimental.pallas import tpu as pltpu
from jax.experimental.pallas import tpu_sc as plsc

NUM_EMBEDDINGS = 1000000
EMBEDDING_DIM = 32
SHARED_DIM = 4
TABLE_DIM = 28
BATCH = 16384

_info = plsc.get_sparse_core_info()
_NC, _NS = _info.num_cores, _info.num_subcores
_NW = _NC * _NS
_L = 16

_CW = 512
_RFULL = 999936
_NCK = _RFULL // _CW
_CPW = 61
_RING = 256
_FLUSH_AT = _RING - _L
_NFLUSH = 72
_DUMMY0 = BATCH
_OUT_ROWS = BATCH + _RING


@functools.partial(
    pl.kernel,
    mesh=plsc.VectorSubcoreMesh(core_axis_name="c", subcore_axis_name="s"),
    out_type=jax.ShapeDtypeStruct((_OUT_ROWS, 128), jnp.float32),
    compiler_params=pltpu.CompilerParams(
        use_tc_tiling_on_sc=True, needs_layout_passes=False
    ),
    scratch_types=[
        pltpu.VMEM((BATCH,), jnp.int32),
        pltpu.VMEM((BATCH,), jnp.int32),
        pltpu.VMEM((BATCH,), jnp.int32),
        pltpu.VMEM((TABLE_DIM, _CW), jnp.float32),
        pltpu.VMEM((_RING, 128), jnp.float32),
        pltpu.VMEM((_NFLUSH * 2, 128), jnp.int32),
        pltpu.VMEM((_L,), jnp.float32),
    ],
)
def _embed_lookup(tableT_hbm, idx_hbm, tailT_hbm, pat_hbm, out_hbm,
                  xall_v, rbuf_v, bbuf_v, chunk_v, ring_v, obuf_v, pat_v):
    wid = lax.axis_index("s") * _NC + lax.axis_index("c")
    pltpu.sync_copy(idx_hbm, xall_v)
    pltpu.sync_copy(pat_hbm, pat_v)

    lanes = lax.iota(jnp.int32, _L)
    upper_rows = jnp.minimum(lanes + _L, TABLE_DIM - 1)
    is_table_col = lanes < (TABLE_DIM - _L)
    pat = pat_v[...]

    rlo = wid * _CPW * _CW
    rhi = jnp.where(wid == _NW - 1, NUM_EMBEDDINGS, rlo + _CPW * _CW)

    def bin_scan(k, cnt):
        v = xall_v[pl.ds(k * _L, _L)]
        m = (v >= rlo) & (v < rhi)
        plsc.store_compressed(rbuf_v.at[pl.ds(cnt, _L)], v, mask=m)
        plsc.store_compressed(bbuf_v.at[pl.ds(cnt, _L)], k * _L + lanes,
                              mask=m)
        return cnt + plsc.all_reduce_population_count(m)[0]

    npairs = lax.fori_loop(0, BATCH // _L, bin_scan, jnp.int32(0))
    jvregs = (npairs + _L - 1) // _L

    def flush(ocnt, fj):
        osplat = jnp.full((_L,), ocnt, jnp.int32)
        fbase = fj * 2
        for z in range(_RING // _L):
            pos = z * _L + lanes
            plsc.store_scatter(obuf_v, [fbase + pos // 128, pos % 128],
                               _DUMMY0 + pos, mask=(pos >= osplat))
        for z in range(_RING // 128):
            pltpu.sync_copy(
                ring_v.at[pl.ds(z * 128, 128)],
                out_hbm.at[obuf_v.at[fj * 2 + z]],
            )
        return jnp.int32(0), fj + 1

    def no_flush(ocnt, fj):
        return ocnt, fj

    def make_jloop(cbase, cwidth):
        def j_body(j, st):
            ocnt, fj = st
            ocnt, fj = lax.cond(ocnt >= _FLUSH_AT, flush, no_flush, ocnt, fj)
            v = rbuf_v[pl.ds(j * _L, _L)]
            act = (j * _L + lanes) < npairs
            m = act & (v >= cbase) & (v < cbase + cwidth)
            c = plsc.all_reduce_population_count(m)[0]

            def w_body(s):
                t, m_, o_ = s
                ffs = plsc.all_reduce_ffs(m_)
                p_vec = j * _L + ffs
                rsp = plsc.load_gather(rbuf_v, [p_vec])
                bsp = plsc.load_gather(bbuf_v, [p_vec])
                col = rsp - cbase
                lo = plsc.load_gather(chunk_v, [lanes, col])
                gu = plsc.load_gather(chunk_v, [upper_rows, col])
                hi = jnp.where(is_table_col, gu, pat)
                ring_v[o_, pl.ds(0, _L)] = lo
                ring_v[o_, pl.ds(_L, _L)] = hi
                plsc.store_scatter(
                    obuf_v,
                    [jnp.full((_L,), fj * 2 + o_ // 128, jnp.int32),
                     jnp.full((_L,), o_ % 128, jnp.int32)],
                    bsp, mask=(lanes == 0))
                m_ = m_ & (lanes != ffs)
                return t + 1, m_, o_ + 1

            _, _, ocnt = lax.while_loop(lambda s: s[0] < c, w_body,
                                        (jnp.int32(0), m, ocnt))
            return ocnt, fj
        return j_body

    def chunk_body(ck, st):
        cbase = (wid * _CPW + ck) * _CW
        off = pl.multiple_of(cbase, _CW)
        pltpu.sync_copy(tableT_hbm.at[:, pl.ds(off, _CW)], chunk_v)
        return lax.fori_loop(0, jvregs, make_jloop(cbase, _CW), st)

    nck = _CPW + jnp.where(wid == _NW - 1, 1, 0)
    st = lax.fori_loop(0, nck, chunk_body, (jnp.int32(0), jnp.int32(0)))

    def tail_fn(st):
        pltpu.sync_copy(tailT_hbm, chunk_v.at[:, pl.ds(0, 128)])
        return lax.fori_loop(0, jvregs,
                             make_jloop(jnp.int32(_RFULL), 128), st)

    st = lax.cond(wid == _NW - 1, tail_fn, lambda s: s, st)

    flush(st[0], st[1])


def kernel(x, embed_table, shared_embed):
    idx = x.astype(jnp.int32)
    tableT = embed_table.T
    tailT = jnp.pad(tableT[:, _RFULL:], ((0, 0), (0, 128 - 64)))
    pat16 = jnp.concatenate(
        [jnp.zeros((_L - SHARED_DIM,), jnp.float32),
         shared_embed.reshape(SHARED_DIM).astype(jnp.float32)]
    )
    out = _embed_lookup(tableT, idx, tailT, pat16)
    return out[:BATCH, :EMBEDDING_DIM].reshape(BATCH, 1, EMBEDDING_DIM)
